# baseline (speedup 1.0000x reference)
import jax
import jax.numpy as jnp
import numpy as np
from jax import lax
from jax.experimental import pallas as pl
from jax.experimental.pallas import tpu as pltpu
from jax.experimental.pallas import tpu_sc as plsc

N_NODES = 10000
N_PAD = 10240
N_EDGES = 320000
HIDDEN = 128
NUM_RBF = 16
CUTOFF_UPPER = 5.0

NC = 2
NS = 16
NW = NC * NS
EDGES_PER_W = N_EDGES // NW
EB = 40
NBLK = EDGES_PER_W // EB
ROWS_PER_TILE = N_PAD // NS


def _silu(v):
    return v * (1.0 / (1.0 + jnp.exp(-v)))


def _filter_body(eat_ref, w1t_ref, b1_ref, w2t_ref, b2_ref, o_ref):
    eat = eat_ref[...]
    h = lax.dot_general(eat, w1t_ref[...], (((0,), (0,)), ((), ())),
                        preferred_element_type=jnp.float32)
    h = h + b1_ref[...]
    h = _silu(h)
    w = jnp.dot(h, w2t_ref[...], preferred_element_type=jnp.float32)
    o_ref[...] = w + b2_ref[...]


def _filter_w(eat, w1t, b1, w2t, b2):
    BE = 3200
    grid = N_EDGES // BE
    return pl.pallas_call(
        _filter_body,
        grid=(grid,),
        in_specs=[
            pl.BlockSpec((NUM_RBF, BE), lambda i: (0, i)),
            pl.BlockSpec((NUM_RBF, HIDDEN), lambda i: (0, 0)),
            pl.BlockSpec((1, HIDDEN), lambda i: (0, 0)),
            pl.BlockSpec((HIDDEN, HIDDEN), lambda i: (0, 0)),
            pl.BlockSpec((1, HIDDEN), lambda i: (0, 0)),
        ],
        out_specs=pl.BlockSpec((BE, HIDDEN), lambda i: (i, 0)),
        out_shape=jax.ShapeDtypeStruct((N_EDGES, HIDDEN), jnp.float32),
    )(eat, w1t, b1, w2t, b2)


def _cutoff_body(ew_ref, c_ref):
    ew = ew_ref[...]
    c = 0.5 * (jnp.cos(ew * (np.pi / CUTOFF_UPPER)) + 1.0)
    c_ref[...] = c * (ew < CUTOFF_UPPER).astype(jnp.float32)


def _cutoff(edge_weight):
    return pl.pallas_call(
        _cutoff_body,
        out_shape=jax.ShapeDtypeStruct((N_EDGES,), jnp.float32),
    )(edge_weight)


def _xh_body(x_ref, wt_ref, o_ref):
    o_ref[...] = jnp.dot(x_ref[...], wt_ref[...],
                         preferred_element_type=jnp.float32)


def _xh(x, lin1t):
    BN = 2000
    grid = N_NODES // BN
    return pl.pallas_call(
        _xh_body,
        grid=(grid,),
        in_specs=[
            pl.BlockSpec((BN, HIDDEN), lambda i: (i, 0)),
            pl.BlockSpec((HIDDEN, HIDDEN), lambda i: (0, 0)),
        ],
        out_specs=pl.BlockSpec((BN, HIDDEN), lambda i: (i, 0)),
        out_shape=jax.ShapeDtypeStruct((N_NODES, HIDDEN), jnp.float32),
    )(x, lin1t)


def _sc_body(xh_hbm, w_hbm, c_hbm, src_hbm, dst_hbm, out_hbm, cnt_hbm,
             acc, cnt, sv0, sv1, dv0, dv1, cv0, cv1, xj0, xj1, wv, msg,
             ones_v, zbuf, zcnt, si0, si1, sg0, sg1, sem_w):
    c = lax.axis_index("c")
    s = lax.axis_index("s")
    wid = c * NS + s

    @pl.loop(0, 128)
    def _zb(j):
        for k in range(8):
            zbuf[j, pl.ds(k * 16, 16)] = jnp.zeros((16,), jnp.float32)

    @pl.loop(0, ROWS_PER_TILE // 16)
    def _zc(j):
        zcnt[pl.ds(j * 16, 16)] = jnp.zeros((16,), jnp.float32)

    for o in (0, 16, EB - 16):
        ones_v[pl.ds(o, 16)] = jnp.ones((16,), jnp.float32)

    for b in range(ROWS_PER_TILE // 128):
        pltpu.sync_copy(zbuf, acc.at[pl.ds(s * ROWS_PER_TILE + b * 128, 128)])
    pltpu.sync_copy(zcnt, cnt.at[pl.ds(s * ROWS_PER_TILE, ROWS_PER_TILE)])

    plsc.subcore_barrier()

    base = wid * EDGES_PER_W

    idx_a = (sv0, dv0, cv0, si0)
    idx_b = (sv1, dv1, cv1, si1)

    def issue_idx(i, ib):
        sv, dv, cv, si = ib
        off = base + i * EB
        pltpu.async_copy(src_hbm.at[pl.ds(off, EB)], sv, si)
        pltpu.async_copy(dst_hbm.at[pl.ds(off, EB)], dv, si)
        pltpu.async_copy(c_hbm.at[pl.ds(off, EB)], cv.at[pl.ds(0, EB)], si)

    def wait_idx(ib):
        sv, dv, cv, si = ib
        pltpu.make_async_copy(src_hbm.at[pl.ds(0, EB)], sv, si).wait()
        pltpu.make_async_copy(dst_hbm.at[pl.ds(0, EB)], dv, si).wait()
        pltpu.make_async_copy(c_hbm.at[pl.ds(0, EB)],
                              cv.at[pl.ds(0, EB)], si).wait()

    def mul_block(i, xj, ib):
        sv, dv, cv, _ = ib
        off = i * EB

        @pl.loop(0, EB)
        def _mul(j):
            cj = cv[pl.ds(j, 16)][0]
            for k in range(8):
                sl = pl.ds(k * 16, 16)
                msg[j, sl] = xj[j, sl] * wv[j, sl] * cj

        pltpu.sync_copy(msg, acc.at[dv], add=True)
        pltpu.sync_copy(ones_v, cnt.at[dv], add=True)

    issue_idx(0, idx_a)
    issue_idx(1, idx_b)
    wait_idx(idx_a)
    g0 = pltpu.async_copy(xh_hbm.at[sv0], xj0, sg0)
    w0 = pltpu.async_copy(w_hbm.at[pl.ds(base, EB)], wv, sem_w)
    g0.wait()
    w0.wait()

    @pl.loop(0, NBLK // 2 - 1)
    def _pair(p):
        wait_idx(idx_b)
        gb = pltpu.async_copy(xh_hbm.at[sv1], xj1, sg1)
        mul_block(2 * p, xj0, idx_a)
        wb = pltpu.async_copy(w_hbm.at[pl.ds(base + (2 * p + 1) * EB, EB)],
                              wv, sem_w)
        issue_idx(2 * p + 2, idx_a)
        gb.wait()
        wb.wait()
        wait_idx(idx_a)
        ga = pltpu.async_copy(xh_hbm.at[sv0], xj0, sg0)
        mul_block(2 * p + 1, xj1, idx_b)
        wa = pltpu.async_copy(w_hbm.at[pl.ds(base + (2 * p + 2) * EB, EB)],
                              wv, sem_w)
        issue_idx(2 * p + 3, idx_b)
        ga.wait()
        wa.wait()

    wait_idx(idx_b)
    gb = pltpu.async_copy(xh_hbm.at[sv1], xj1, sg1)
    mul_block(NBLK - 2, xj0, idx_a)
    wb = pltpu.async_copy(w_hbm.at[pl.ds(base + (NBLK - 1) * EB, EB)],
                          wv, sem_w)
    gb.wait()
    wb.wait()
    mul_block(NBLK - 1, xj1, idx_b)

    plsc.subcore_barrier()
    pltpu.sync_copy(acc.at[pl.ds(s * ROWS_PER_TILE, ROWS_PER_TILE)],
                    out_hbm.at[c, pl.ds(s * ROWS_PER_TILE, ROWS_PER_TILE)])
    pltpu.sync_copy(cnt.at[pl.ds(s * ROWS_PER_TILE, ROWS_PER_TILE)],
                    cnt_hbm.at[c, pl.ds(s * ROWS_PER_TILE, ROWS_PER_TILE)])


def _sc_msg(xh, w, cearr, src, dst):
    mesh = plsc.VectorSubcoreMesh(core_axis_name="c", subcore_axis_name="s",
                                  num_cores=NC, num_subcores=NS)
    fn = pl.kernel(
        _sc_body,
        out_type=[
            jax.ShapeDtypeStruct((NC, N_PAD, HIDDEN), jnp.float32),
            jax.ShapeDtypeStruct((NC, N_PAD), jnp.float32),
        ],
        mesh=mesh,
        scratch_types=[
            pltpu.VMEM_SHARED((N_PAD, HIDDEN), jnp.float32),
            pltpu.VMEM_SHARED((N_PAD,), jnp.float32),
            pltpu.VMEM((EB,), jnp.int32),
            pltpu.VMEM((EB,), jnp.int32),
            pltpu.VMEM((EB,), jnp.int32),
            pltpu.VMEM((EB,), jnp.int32),
            pltpu.VMEM((EB + 16,), jnp.float32),
            pltpu.VMEM((EB + 16,), jnp.float32),
            pltpu.VMEM((EB, HIDDEN), jnp.float32),
            pltpu.VMEM((EB, HIDDEN), jnp.float32),
            pltpu.VMEM((EB, HIDDEN), jnp.float32),
            pltpu.VMEM((EB, HIDDEN), jnp.float32),
            pltpu.VMEM((EB,), jnp.float32),
            pltpu.VMEM((128, HIDDEN), jnp.float32),
            pltpu.VMEM((ROWS_PER_TILE,), jnp.float32),
            pltpu.SemaphoreType.DMA,
            pltpu.SemaphoreType.DMA,
            pltpu.SemaphoreType.DMA,
            pltpu.SemaphoreType.DMA,
            pltpu.SemaphoreType.DMA,
        ],
    )
    return fn(xh, w, cearr, src, dst)


def _tail_body(agg_ref, cnt_ref, l2t_ref, l2b_ref, lt_ref, lb_ref, o_ref):
    ssum = agg_ref[0] + agg_ref[1]
    cnt = cnt_ref[...]
    denom = jnp.where(cnt > 0, cnt, 1.0)
    agg = ssum / denom
    t = jnp.dot(agg, l2t_ref[...], preferred_element_type=jnp.float32)
    t = t + l2b_ref[...]
    t = _silu(t)
    o = jnp.dot(t, lt_ref[...], preferred_element_type=jnp.float32)
    o_ref[...] = o + lb_ref[...]


def _tail(agg2, cnt1, lin2t, lin2_b2, lint, lin_b2):
    BN = 2000
    grid = N_NODES // BN
    return pl.pallas_call(
        _tail_body,
        grid=(grid,),
        in_specs=[
            pl.BlockSpec((NC, BN, HIDDEN), lambda i: (0, i, 0)),
            pl.BlockSpec((BN, 1), lambda i: (i, 0)),
            pl.BlockSpec((HIDDEN, HIDDEN), lambda i: (0, 0)),
            pl.BlockSpec((1, HIDDEN), lambda i: (0, 0)),
            pl.BlockSpec((HIDDEN, HIDDEN), lambda i: (0, 0)),
            pl.BlockSpec((1, HIDDEN), lambda i: (0, 0)),
        ],
        out_specs=pl.BlockSpec((BN, HIDDEN), lambda i: (i, 0)),
        out_shape=jax.ShapeDtypeStruct((N_NODES, HIDDEN), jnp.float32),
    )(agg2, cnt1, lin2t, lin2_b2, lint, lin_b2)


def kernel(x, edge_index, edge_weight, edge_attr,
           mlp_w1, mlp_b1, mlp_w2, mlp_b2,
           lin1_w, lin2_w, lin2_b, lin_w, lin_b):
    src = edge_index[0].astype(jnp.int32)
    dst = edge_index[1].astype(jnp.int32)

    eat = edge_attr.T
    w1t = mlp_w1.T
    w2t = mlp_w2.T
    lin1t = lin1_w.T
    lin2t = lin2_w.T
    lint = lin_w.T
    b1 = mlp_b1.reshape(1, HIDDEN)
    b2 = mlp_b2.reshape(1, HIDDEN)
    lin2_b2 = lin2_b.reshape(1, HIDDEN)
    lin_b2 = lin_b.reshape(1, HIDDEN)

    w = _filter_w(eat, w1t, b1, w2t, b2)
    cearr = _cutoff(edge_weight)
    xh = _xh(x, lin1t)

    agg2, cnt2 = _sc_msg(xh, w, cearr, src, dst)

    cnt1 = (cnt2[0] + cnt2[1])[:N_NODES].reshape(N_NODES, 1)
    return _tail(agg2, cnt1, lin2t, lin2_b2, lint, lin_b2)
